# baseline (speedup 1.0000x reference)
import functools

import jax
import jax.numpy as jnp
from jax import lax
from jax.experimental import pallas as pl
from jax.experimental.pallas import tpu as pltpu
from jax.experimental.pallas import tpu_sc as plsc

N_NODES = 10000
N_ACC = 10240
NUM_CORES = 2
NUM_SUBCORES = 16
NW = NUM_CORES * NUM_SUBCORES
K_EDGES = 128
ROWS_PER_TILE = N_ACC // NUM_SUBCORES

_HIGH = jax.lax.Precision.HIGHEST


def _dot(a, b):
    return jax.lax.dot_general(a, b, (((1,), (0,)), ((), ())),
                               precision=_HIGH,
                               preferred_element_type=jnp.float32)


N_BUF = 4


def _sc_segment_sum(table, gidx, sidx, zeros, d, n_chunks, col_split):
    mesh = plsc.VectorSubcoreMesh(core_axis_name="c", subcore_axis_name="s")
    assert n_chunks % N_BUF == 0
    n_groups = n_chunks // N_BUF

    @functools.partial(
        pl.kernel,
        out_type=jax.ShapeDtypeStruct((NUM_CORES, N_ACC, d), jnp.float32),
        mesh=mesh,
        scratch_types=[
            pltpu.VMEM((n_chunks, K_EDGES), jnp.int32),
            pltpu.VMEM((n_chunks, K_EDGES), jnp.int32),
        ] + [pltpu.VMEM((K_EDGES, d), jnp.float32) for _ in range(N_BUF)]
          + [pltpu.VMEM_SHARED((N_ACC, d), jnp.float32)]
          + [pltpu.SemaphoreType.DMA for _ in range(N_BUF)],
        compiler_params=pltpu.CompilerParams(use_tc_tiling_on_sc=False),
    )
    def seg_kernel(table_hbm, gidx_hbm, sidx_hbm, zeros_hbm, out_hbm,
                   gidx_v, sidx_v, *rest):
        rows = rest[:N_BUF]
        acc_sh = rest[N_BUF]
        gsem = rest[N_BUF + 1:]
        c = lax.axis_index("c")
        s = lax.axis_index("s")
        if col_split:
            wid = s
            tbl = table_hbm.at[c]
        else:
            wid = s * NUM_CORES + c
            tbl = table_hbm
        pltpu.sync_copy(zeros_hbm.at[pl.ds(s * ROWS_PER_TILE, ROWS_PER_TILE)],
                        acc_sh.at[pl.ds(s * ROWS_PER_TILE, ROWS_PER_TILE)])
        pltpu.sync_copy(gidx_hbm.at[wid], gidx_v)
        pltpu.sync_copy(sidx_hbm.at[wid], sidx_v)
        plsc.subcore_barrier()

        for b in range(N_BUF):
            pltpu.make_async_copy(tbl.at[gidx_v.at[b]], rows[b],
                                  gsem[b]).start()

        @pl.loop(0, n_groups - 1)
        def _(g):
            for b in range(N_BUF):
                chunk = g * N_BUF + b
                pltpu.make_async_copy(tbl.at[gidx_v.at[chunk]], rows[b],
                                      gsem[b]).wait()
                pltpu.sync_copy(rows[b], acc_sh.at[sidx_v.at[chunk]], add=True)
                pltpu.make_async_copy(
                    tbl.at[gidx_v.at[chunk + N_BUF]], rows[b],
                    gsem[b]).start()

        for b in range(N_BUF):
            chunk = (n_groups - 1) * N_BUF + b
            pltpu.make_async_copy(tbl.at[gidx_v.at[chunk]], rows[b],
                                  gsem[b]).wait()
            pltpu.sync_copy(rows[b], acc_sh.at[sidx_v.at[chunk]], add=True)

        plsc.subcore_barrier()
        pltpu.sync_copy(acc_sh.at[pl.ds(s * ROWS_PER_TILE, ROWS_PER_TILE)],
                        out_hbm.at[c, pl.ds(s * ROWS_PER_TILE, ROWS_PER_TILE)])

    return seg_kernel(table, gidx, sidx, zeros)


ROW_BLK = 1000


def _tc_project(x, Ws, Wn):
    n, d_in = x.shape
    d_s = Ws.shape[1]
    d_n = Wn.shape[1]

    def body(x_ref, ws_ref, wn_ref, hs_ref, hw_ref):
        xb = x_ref[...]
        hs_ref[...] = _dot(xb, ws_ref[...])
        hw_ref[...] = _dot(xb, wn_ref[...])

    return pl.pallas_call(
        body,
        grid=(n // ROW_BLK,),
        in_specs=[
            pl.BlockSpec((ROW_BLK, d_in), lambda i: (i, 0)),
            pl.BlockSpec((d_in, d_s), lambda i: (0, 0)),
            pl.BlockSpec((d_in, d_n), lambda i: (0, 0)),
        ],
        out_specs=[
            pl.BlockSpec((ROW_BLK, d_s), lambda i: (i, 0)),
            pl.BlockSpec((ROW_BLK, d_n), lambda i: (i, 0)),
        ],
        out_shape=[
            jax.ShapeDtypeStruct((n, d_s), jnp.float32),
            jax.ShapeDtypeStruct((n, d_n), jnp.float32),
        ],
    )(x, Ws, Wn)


def _tc_combine_project(hs, acc, deg, b, Ws_next, Wn_next, acc_concat):
    n, d = hs.shape
    d_s = Ws_next.shape[1]
    d_n = Wn_next.shape[1]
    d_acc = d // 2 if acc_concat else d

    def body(hs_ref, acc_ref, deg_ref, b_ref, ws_ref, wn_ref, hs2_ref, hw2_ref):
        degv = deg_ref[0, :, :1] + deg_ref[1, :, :1]
        inv = 1.0 / jnp.maximum(degv, 1.0)
        if acc_concat:
            neigh = jnp.concatenate([acc_ref[0], acc_ref[1]], axis=-1) * inv
        else:
            neigh = (acc_ref[0] + acc_ref[1]) * inv
        h = jnp.maximum(hs_ref[...] + neigh + b_ref[...], 0.0)
        hs2_ref[...] = _dot(h, ws_ref[...])
        hw2_ref[...] = _dot(h, wn_ref[...])

    return pl.pallas_call(
        body,
        grid=(n // ROW_BLK,),
        in_specs=[
            pl.BlockSpec((ROW_BLK, d), lambda i: (i, 0)),
            pl.BlockSpec((2, ROW_BLK, d_acc), lambda i: (0, i, 0)),
            pl.BlockSpec((2, ROW_BLK, 16), lambda i: (0, i, 0)),
            pl.BlockSpec((1, d), lambda i: (0, 0)),
            pl.BlockSpec((d, d_s), lambda i: (0, 0)),
            pl.BlockSpec((d, d_n), lambda i: (0, 0)),
        ],
        out_specs=[
            pl.BlockSpec((ROW_BLK, d_s), lambda i: (i, 0)),
            pl.BlockSpec((ROW_BLK, d_n), lambda i: (i, 0)),
        ],
        out_shape=[
            jax.ShapeDtypeStruct((n, d_s), jnp.float32),
            jax.ShapeDtypeStruct((n, d_n), jnp.float32),
        ],
    )(hs, acc, deg, b, Ws_next, Wn_next)


def _tc_final(hs3, acc3, deg, b3, pW1, pb1, pW2, pb2, pW3, pb3):
    n, d = hs3.shape
    a_dim = pW3.shape[1]

    def body(hs_ref, acc_ref, deg_ref, b_ref, w1_ref, b1_ref, w2_ref, b2_ref,
             w3_ref, b3_ref, out_ref):
        degv = deg_ref[0, :, :1] + deg_ref[1, :, :1]
        inv = 1.0 / jnp.maximum(degv, 1.0)
        neigh = (acc_ref[0] + acc_ref[1]) * inv
        h = jnp.maximum(hs_ref[...] + neigh + b_ref[...], 0.0)
        g = jnp.sum(h, axis=0, keepdims=True) * (1.0 / n)
        l1 = jnp.maximum(_dot(g, w1_ref[...]) + b1_ref[...], 0.0)
        l2 = jnp.maximum(_dot(l1, w2_ref[...]) + b2_ref[...], 0.0)
        out_ref[...] = _dot(l2, w3_ref[...]) + b3_ref[...]

    return pl.pallas_call(
        body,
        out_shape=jax.ShapeDtypeStruct((1, a_dim), jnp.float32),
    )(hs3, acc3, deg, b3, pW1, pb1, pW2, pb2, pW3, pb3)


def kernel(x, edge_index, Ws1, Wn1, b1, Ws2, Wn2, b2, Ws3, Wn3, b3,
           pW1, pb1, pW2, pb2, pW3, pb3):
    n = x.shape[0]
    e = edge_index.shape[1]
    src = edge_index[0]
    dst = edge_index[1]

    n_chunks = -(-e // (NW * K_EDGES))
    n_chunks = -(-n_chunks // N_BUF) * N_BUF
    e_pad = NW * K_EDGES * n_chunks
    pad = e_pad - e
    pad_rows = n + (jnp.arange(pad, dtype=jnp.int32) % (N_ACC - n))
    src_flat = jnp.concatenate([src, jnp.zeros((pad,), jnp.int32)])
    dsts_flat = jnp.concatenate([dst, pad_rows])
    dstg_flat = jnp.concatenate([dst, jnp.zeros((pad,), jnp.int32)])
    idx_r = (NW, n_chunks, K_EDGES)
    src_r = src_flat.reshape(idx_r)
    dsts_r = dsts_flat.reshape(idx_r)
    dstg_r = dstg_flat.reshape(idx_r)
    idx_c = (NUM_SUBCORES, 2 * n_chunks, K_EDGES)
    src_c = src_flat.reshape(idx_c)
    dsts_c = dsts_flat.reshape(idx_c)

    zeros64 = jnp.zeros((N_ACC, 64), jnp.float32)
    zeros16 = jnp.zeros((N_ACC, 16), jnp.float32)
    ones16 = jnp.ones((n, 16), jnp.float32)

    deg = _sc_segment_sum(ones16, dstg_r, dsts_r, zeros16, 16, n_chunks,
                          col_split=False)
    deg = deg[:, :n]

    b1r = b1.reshape(1, -1)
    b2r = b2.reshape(1, -1)
    b3r = b3.reshape(1, -1)

    hs1, hw1 = _tc_project(x, Ws1, Wn1)
    hw1h = jnp.stack([hw1[:, :64], hw1[:, 64:]])
    acc1 = _sc_segment_sum(hw1h, src_c, dsts_c, zeros64, 64, 2 * n_chunks,
                           col_split=True)[:, :n]
    hs2, hw2 = _tc_combine_project(hs1, acc1, deg, b1r, Ws2, Wn2,
                                   acc_concat=True)
    hw2h = jnp.stack([hw2[:, :64], hw2[:, 64:]])
    acc2 = _sc_segment_sum(hw2h, src_c, dsts_c, zeros64, 64, 2 * n_chunks,
                           col_split=True)[:, :n]
    hs3, hw3 = _tc_combine_project(hs2, acc2, deg, b2r, Ws3, Wn3,
                                   acc_concat=True)
    acc3 = _sc_segment_sum(hw3, src_r, dsts_r, zeros64, 64, n_chunks,
                           col_split=False)[:, :n]
    logits = _tc_final(hs3, acc3, deg, b3r,
                       pW1, pb1.reshape(1, -1), pW2, pb2.reshape(1, -1),
                       pW3, pb3.reshape(1, -1))
    return logits[0]


# speedup vs baseline: 1.5739x; 1.5739x over previous
import functools

import jax
import jax.numpy as jnp
from jax import lax
from jax.experimental import pallas as pl
from jax.experimental.pallas import tpu as pltpu
from jax.experimental.pallas import tpu_sc as plsc

N_NODES = 10000
N_ACC = 10240
NUM_CORES = 2
NUM_SUBCORES = 16
NW = NUM_CORES * NUM_SUBCORES
K_EDGES = 128
ROWS_PER_TILE = N_ACC // NUM_SUBCORES

_HIGH = jax.lax.Precision.HIGHEST


def _dot(a, b):
    return jax.lax.dot_general(a, b, (((1,), (0,)), ((), ())),
                               precision=_HIGH,
                               preferred_element_type=jnp.float32)


CPB = 4


def _sc_segment_sum(table, gidx, sidx, zeros, d, n_chunks, col_split):
    mesh = plsc.VectorSubcoreMesh(core_axis_name="c", subcore_axis_name="s")
    n_blocks = n_chunks // CPB
    assert n_chunks % (2 * CPB) == 0

    @functools.partial(
        pl.kernel,
        out_type=jax.ShapeDtypeStruct((NUM_CORES, N_ACC, d), jnp.float32),
        mesh=mesh,
        scratch_types=(
            [pltpu.VMEM((CPB, K_EDGES), jnp.int32) for _ in range(4)]
            + [pltpu.VMEM((K_EDGES, d), jnp.float32) for _ in range(CPB)]
            + [pltpu.VMEM_SHARED((N_ACC, d), jnp.float32),
               pltpu.VMEM_SHARED((N_ACC, d), jnp.float32)]
            + [pltpu.SemaphoreType.DMA for _ in range(CPB + 2)]
        ),
        compiler_params=pltpu.CompilerParams(use_tc_tiling_on_sc=False),
    )
    def seg_kernel(table_hbm, gidx_hbm, sidx_hbm, zeros_hbm, out_hbm, *rest):
        gbuf = rest[0:2]
        dbuf = rest[2:4]
        rows = rest[4:4 + CPB]
        acc_sh = rest[4 + CPB]
        tbl_sh = rest[5 + CPB]
        gsem = rest[6 + CPB:6 + 2 * CPB]
        isem = rest[6 + 2 * CPB:]
        c = lax.axis_index("c")
        s = lax.axis_index("s")
        if col_split:
            wid = s
            tbl = table_hbm.at[c]
        else:
            wid = s * NUM_CORES + c
            tbl = table_hbm
        rslc = pl.ds(s * ROWS_PER_TILE, ROWS_PER_TILE)
        pltpu.sync_copy(zeros_hbm.at[rslc], acc_sh.at[rslc])
        pltpu.sync_copy(tbl.at[rslc], tbl_sh.at[rslc])
        pltpu.sync_copy(gidx_hbm.at[wid, pl.ds(0, CPB)], gbuf[0])
        pltpu.sync_copy(sidx_hbm.at[wid, pl.ds(0, CPB)], dbuf[0])
        pltpu.make_async_copy(gidx_hbm.at[wid, pl.ds(CPB, CPB)], gbuf[1],
                              isem[1]).start()
        pltpu.make_async_copy(sidx_hbm.at[wid, pl.ds(CPB, CPB)], dbuf[1],
                              isem[1]).start()
        plsc.subcore_barrier()

        for j in range(CPB):
            pltpu.make_async_copy(tbl_sh.at[gbuf[0].at[j]], rows[j],
                                  gsem[j]).start()

        def half_step(t, p, blk_off):
            q = 1 - p
            pltpu.make_async_copy(gidx_hbm.at[wid, pl.ds(0, CPB)], gbuf[q],
                                  isem[q]).wait()
            pltpu.make_async_copy(sidx_hbm.at[wid, pl.ds(0, CPB)], dbuf[q],
                                  isem[q]).wait()
            for j in range(CPB):
                pltpu.make_async_copy(tbl_sh.at[gbuf[p].at[j]], rows[j],
                                      gsem[j]).wait()
                pltpu.sync_copy(rows[j], acc_sh.at[dbuf[p].at[j]], add=True)
                pltpu.make_async_copy(tbl_sh.at[gbuf[q].at[j]], rows[j],
                                      gsem[j]).start()
            nxt = pl.ds((blk_off + 2) * CPB, CPB)
            pltpu.make_async_copy(gidx_hbm.at[wid, nxt], gbuf[p],
                                  isem[p]).start()
            pltpu.make_async_copy(sidx_hbm.at[wid, nxt], dbuf[p],
                                  isem[p]).start()

        @pl.loop(0, n_blocks // 2)
        def _(t):
            half_step(t, 0, 2 * t)
            half_step(t, 1, 2 * t + 1)

        for j in range(CPB):
            pltpu.make_async_copy(tbl_sh.at[gbuf[0].at[j]], rows[j],
                                  gsem[j]).wait()
        pltpu.make_async_copy(gidx_hbm.at[wid, pl.ds(0, CPB)], gbuf[1],
                              isem[1]).wait()
        pltpu.make_async_copy(sidx_hbm.at[wid, pl.ds(0, CPB)], dbuf[1],
                              isem[1]).wait()

        plsc.subcore_barrier()
        pltpu.sync_copy(acc_sh.at[rslc], out_hbm.at[c, rslc])

    return seg_kernel(table, gidx, sidx, zeros)


ROW_BLK = 1000


def _tc_project(x, Ws, Wn):
    n, d_in = x.shape
    d_s = Ws.shape[1]
    d_n = Wn.shape[1]

    def body(x_ref, ws_ref, wn_ref, hs_ref, hw_ref):
        xb = x_ref[...]
        hs_ref[...] = _dot(xb, ws_ref[...])
        hw_ref[...] = _dot(xb, wn_ref[...])

    return pl.pallas_call(
        body,
        grid=(n // ROW_BLK,),
        in_specs=[
            pl.BlockSpec((ROW_BLK, d_in), lambda i: (i, 0)),
            pl.BlockSpec((d_in, d_s), lambda i: (0, 0)),
            pl.BlockSpec((d_in, d_n), lambda i: (0, 0)),
        ],
        out_specs=[
            pl.BlockSpec((ROW_BLK, d_s), lambda i: (i, 0)),
            pl.BlockSpec((ROW_BLK, d_n), lambda i: (i, 0)),
        ],
        out_shape=[
            jax.ShapeDtypeStruct((n, d_s), jnp.float32),
            jax.ShapeDtypeStruct((n, d_n), jnp.float32),
        ],
    )(x, Ws, Wn)


def _tc_combine_project(hs, acc, deg, b, Ws_next, Wn_next, acc_concat):
    n, d = hs.shape
    d_s = Ws_next.shape[1]
    d_n = Wn_next.shape[1]
    d_acc = d // 2 if acc_concat else d

    def body(hs_ref, acc_ref, deg_ref, b_ref, ws_ref, wn_ref, hs2_ref, hw2_ref):
        degv = deg_ref[0, :, :1] + deg_ref[1, :, :1]
        inv = 1.0 / jnp.maximum(degv, 1.0)
        if acc_concat:
            neigh = jnp.concatenate([acc_ref[0], acc_ref[1]], axis=-1) * inv
        else:
            neigh = (acc_ref[0] + acc_ref[1]) * inv
        h = jnp.maximum(hs_ref[...] + neigh + b_ref[...], 0.0)
        hs2_ref[...] = _dot(h, ws_ref[...])
        hw2_ref[...] = _dot(h, wn_ref[...])

    return pl.pallas_call(
        body,
        grid=(n // ROW_BLK,),
        in_specs=[
            pl.BlockSpec((ROW_BLK, d), lambda i: (i, 0)),
            pl.BlockSpec((2, ROW_BLK, d_acc), lambda i: (0, i, 0)),
            pl.BlockSpec((2, ROW_BLK, 16), lambda i: (0, i, 0)),
            pl.BlockSpec((1, d), lambda i: (0, 0)),
            pl.BlockSpec((d, d_s), lambda i: (0, 0)),
            pl.BlockSpec((d, d_n), lambda i: (0, 0)),
        ],
        out_specs=[
            pl.BlockSpec((ROW_BLK, d_s), lambda i: (i, 0)),
            pl.BlockSpec((ROW_BLK, d_n), lambda i: (i, 0)),
        ],
        out_shape=[
            jax.ShapeDtypeStruct((n, d_s), jnp.float32),
            jax.ShapeDtypeStruct((n, d_n), jnp.float32),
        ],
    )(hs, acc, deg, b, Ws_next, Wn_next)


def _tc_final(hs3, acc3, deg, b3, pW1, pb1, pW2, pb2, pW3, pb3):
    n, d = hs3.shape
    a_dim = pW3.shape[1]

    def body(hs_ref, acc_ref, deg_ref, b_ref, w1_ref, b1_ref, w2_ref, b2_ref,
             w3_ref, b3_ref, out_ref):
        degv = deg_ref[0, :, :1] + deg_ref[1, :, :1]
        inv = 1.0 / jnp.maximum(degv, 1.0)
        neigh = (acc_ref[0] + acc_ref[1]) * inv
        h = jnp.maximum(hs_ref[...] + neigh + b_ref[...], 0.0)
        g = jnp.sum(h, axis=0, keepdims=True) * (1.0 / n)
        l1 = jnp.maximum(_dot(g, w1_ref[...]) + b1_ref[...], 0.0)
        l2 = jnp.maximum(_dot(l1, w2_ref[...]) + b2_ref[...], 0.0)
        out_ref[...] = _dot(l2, w3_ref[...]) + b3_ref[...]

    return pl.pallas_call(
        body,
        out_shape=jax.ShapeDtypeStruct((1, a_dim), jnp.float32),
    )(hs3, acc3, deg, b3, pW1, pb1, pW2, pb2, pW3, pb3)


def kernel(x, edge_index, Ws1, Wn1, b1, Ws2, Wn2, b2, Ws3, Wn3, b3,
           pW1, pb1, pW2, pb2, pW3, pb3):
    n = x.shape[0]
    e = edge_index.shape[1]
    src = edge_index[0]
    dst = edge_index[1]

    n_chunks = -(-e // (NW * K_EDGES))
    n_chunks = -(-n_chunks // (2 * CPB)) * (2 * CPB)
    e_pad = NW * K_EDGES * n_chunks
    pad = e_pad - e
    pad_rows = n + (jnp.arange(pad, dtype=jnp.int32) % (N_ACC - n))
    src_flat = jnp.concatenate([src, jnp.zeros((pad,), jnp.int32)])
    dsts_flat = jnp.concatenate([dst, pad_rows])
    dstg_flat = jnp.concatenate([dst, jnp.zeros((pad,), jnp.int32)])
    ipad = ((0, 0), (0, 2 * CPB), (0, 0))
    idx_r = (NW, n_chunks, K_EDGES)
    src_r = jnp.pad(src_flat.reshape(idx_r), ipad)
    dsts_r = jnp.pad(dsts_flat.reshape(idx_r), ipad)
    dstg_r = jnp.pad(dstg_flat.reshape(idx_r), ipad)
    idx_c = (NUM_SUBCORES, 2 * n_chunks, K_EDGES)
    src_c = jnp.pad(src_flat.reshape(idx_c), ipad)
    dsts_c = jnp.pad(dsts_flat.reshape(idx_c), ipad)

    zeros64 = jnp.zeros((N_ACC, 64), jnp.float32)
    zeros16 = jnp.zeros((N_ACC, 16), jnp.float32)
    ones16 = jnp.ones((N_ACC, 16), jnp.float32)

    deg = _sc_segment_sum(ones16, dstg_r, dsts_r, zeros16, 16, n_chunks,
                          col_split=False)
    deg = deg[:, :n]

    b1r = b1.reshape(1, -1)
    b2r = b2.reshape(1, -1)
    b3r = b3.reshape(1, -1)

    hs1, hw1 = _tc_project(x, Ws1, Wn1)
    tpad = jnp.zeros((N_ACC - n, 64), jnp.float32)
    hw1h = jnp.stack([jnp.concatenate([hw1[:, :64], tpad]), jnp.concatenate([hw1[:, 64:], tpad])])
    acc1 = _sc_segment_sum(hw1h, src_c, dsts_c, zeros64, 64, 2 * n_chunks,
                           col_split=True)[:, :n]
    hs2, hw2 = _tc_combine_project(hs1, acc1, deg, b1r, Ws2, Wn2,
                                   acc_concat=True)
    hw2h = jnp.stack([jnp.concatenate([hw2[:, :64], tpad]), jnp.concatenate([hw2[:, 64:], tpad])])
    acc2 = _sc_segment_sum(hw2h, src_c, dsts_c, zeros64, 64, 2 * n_chunks,
                           col_split=True)[:, :n]
    hs3, hw3 = _tc_combine_project(hs2, acc2, deg, b2r, Ws3, Wn3,
                                   acc_concat=True)
    hw3p = jnp.concatenate([hw3, tpad])
    acc3 = _sc_segment_sum(hw3p, src_r, dsts_r, zeros64, 64, n_chunks,
                           col_split=False)[:, :n]
    logits = _tc_final(hs3, acc3, deg, b3r,
                       pW1, pb1.reshape(1, -1), pW2, pb2.reshape(1, -1),
                       pW3, pb3.reshape(1, -1))
    return logits[0]
